# baseline (speedup 1.0000x reference)
import functools

import jax
import jax.numpy as jnp
from jax import lax
from jax.experimental import pallas as pl
from jax.experimental.pallas import tpu as pltpu
from jax.experimental.pallas import tpu_sc as plsc


BM = 256
KC = 1024
BLK = 128


def _vq_body(nk, x_ref, ct_ref, x2_ref, c2_ref, iotaf_ref, idx_ref, mind_ref,
             cls_ref, clu_ref, *, cpc):
    x = -2.0 * x_ref[...]
    x2 = x2_ref[...]

    m = jnp.full((BM, BLK), jnp.inf, dtype=jnp.float32)
    f = jnp.zeros((BM, BLK), dtype=jnp.float32)

    for j in range(nk):
        ctc = ct_ref[j * KC:(j + 1) * KC, :]
        g = lax.dot_general(x, ctc, (((1,), (1,)), ((), ())),
                            preferred_element_type=jnp.float32)
        for b in range(KC // BLK):
            lo = j * KC + b * BLK
            c2 = c2_ref[:, lo:lo + BLK]
            iota_f = iotaf_ref[:, lo:lo + BLK]
            d2 = (x2 + c2) + g[:, b * BLK:(b + 1) * BLK]
            lt = d2 < m
            m = jnp.where(lt, d2, m)
            f = jnp.where(lt, iota_f, f)

    mv = jnp.min(m, axis=1, keepdims=True)
    big = float(nk * KC)
    li = jnp.min(jnp.where(m == mv, f, big), axis=1, keepdims=True)
    bidx = li.astype(jnp.int32).reshape(BM)
    idx_ref[...] = bidx
    mind_ref[...] = jnp.sqrt(jnp.maximum(mv, 0.0)).reshape(BM)
    cls_ref[...] = bidx // cpc
    clu_ref[...] = bidx % cpc


def _make_sc_gather(k, d, b):
    info = plsc.get_sparse_core_info()
    nc, ns = info.num_cores, info.num_subcores
    nw = nc * ns
    b_per_w = b // nw
    mesh = plsc.VectorSubcoreMesh(core_axis_name="c", subcore_axis_name="s")

    @functools.partial(
        pl.kernel, mesh=mesh,
        compiler_params=pltpu.CompilerParams(use_tc_tiling_on_sc=False),
        out_type=jax.ShapeDtypeStruct((b, d), jnp.float32),
        scratch_types=[
            pltpu.VMEM((b_per_w,), jnp.int32),
            pltpu.VMEM((b_per_w, d), jnp.float32),
            pltpu.SemaphoreType.DMA,
        ],
    )
    def gather_k(table_hbm, idx_hbm, out_hbm, idx_v, rows_v, sem):
        wid = lax.axis_index("s") * nc + lax.axis_index("c")
        base = wid * b_per_w
        pltpu.sync_copy(idx_hbm.at[pl.ds(base, b_per_w)], idx_v)
        pltpu.async_copy(table_hbm.at[idx_v], rows_v, sem).wait()
        pltpu.sync_copy(rows_v, out_hbm.at[pl.ds(base, b_per_w)])

    return gather_k


def kernel(x, labels, centers):
    num_classes, cpc, d = centers.shape
    b = x.shape[0]
    k = num_classes * cpc
    allc = centers.reshape(k, d).astype(x.dtype)
    x2 = jnp.sum(x * x, axis=1, keepdims=True)
    c2 = jnp.sum(allc * allc, axis=1)[None, :]
    iotaf = jnp.arange(k, dtype=jnp.float32)[None, :]
    nb = b // BM
    nk = k // KC

    out_shapes = [
        jax.ShapeDtypeStruct((b,), jnp.int32),
        jax.ShapeDtypeStruct((b,), jnp.float32),
        jax.ShapeDtypeStruct((b,), jnp.int32),
        jax.ShapeDtypeStruct((b,), jnp.int32),
    ]
    grid_spec = pl.GridSpec(
        grid=(nb,),
        in_specs=[
            pl.BlockSpec((BM, d), lambda i: (i, 0)),
            pl.BlockSpec((k, d), lambda i: (0, 0)),
            pl.BlockSpec((BM, 1), lambda i: (i, 0)),
            pl.BlockSpec((1, k), lambda i: (0, 0)),
            pl.BlockSpec((1, k), lambda i: (0, 0)),
        ],
        out_specs=[
            pl.BlockSpec((BM,), lambda i: (i,)),
            pl.BlockSpec((BM,), lambda i: (i,)),
            pl.BlockSpec((BM,), lambda i: (i,)),
            pl.BlockSpec((BM,), lambda i: (i,)),
        ],
    )
    idx, mind, cls, clu = pl.pallas_call(
        functools.partial(_vq_body, nk, cpc=cpc),
        grid_spec=grid_spec,
        out_shape=out_shapes,
    )(x, allc, x2, c2, iotaf)

    quant = _make_sc_gather(k, d, b)(allc, idx)

    return (quant, clu, mind, cls, centers, labels)


# speedup vs baseline: 1.0559x; 1.0559x over previous
import functools

import jax
import jax.numpy as jnp
from jax import lax
from jax.experimental import pallas as pl
from jax.experimental.pallas import tpu as pltpu
from jax.experimental.pallas import tpu_sc as plsc


BM = 256
KC = 1024
BLK = 128


def _vq_body(nk, x_ref, ct_ref, x2_ref, c2_ref, iotaf_ref, idx_ref, mind_ref,
             cls_ref, clu_ref, *, cpc):
    x = -2.0 * x_ref[...]
    x2 = x2_ref[...]

    m = jnp.full((BM, BLK), jnp.inf, dtype=jnp.float32)
    f = jnp.zeros((BM, BLK), dtype=jnp.float32)

    for j in range(nk):
        ctc = ct_ref[j * KC:(j + 1) * KC, :]
        g = lax.dot_general(x, ctc, (((1,), (1,)), ((), ())),
                            preferred_element_type=jnp.float32)
        for b in range(KC // BLK):
            lo = j * KC + b * BLK
            c2 = c2_ref[:, lo:lo + BLK]
            iota_f = iotaf_ref[:, lo:lo + BLK]
            d2 = (x2 + c2) + g[:, b * BLK:(b + 1) * BLK]
            lt = d2 < m
            m = jnp.where(lt, d2, m)
            f = jnp.where(lt, iota_f, f)

    mv = jnp.min(m, axis=1, keepdims=True)
    big = float(nk * KC)
    li = jnp.min(jnp.where(m == mv, f, big), axis=1, keepdims=True)
    bidx = li.astype(jnp.int32)
    idx_ref[...] = bidx
    mind_ref[...] = jnp.sqrt(jnp.maximum(mv, 0.0))
    cls_ref[...] = bidx // cpc
    clu_ref[...] = bidx % cpc


def _make_sc_gather(k, d, b):
    info = plsc.get_sparse_core_info()
    nc, ns = info.num_cores, info.num_subcores
    nw = nc * ns
    b_per_w = b // nw
    mesh = plsc.VectorSubcoreMesh(core_axis_name="c", subcore_axis_name="s")

    @functools.partial(
        pl.kernel, mesh=mesh,
        compiler_params=pltpu.CompilerParams(use_tc_tiling_on_sc=False),
        out_type=jax.ShapeDtypeStruct((b, d), jnp.float32),
        scratch_types=[
            pltpu.VMEM((b_per_w,), jnp.int32),
            pltpu.VMEM((b_per_w, d), jnp.float32),
            pltpu.SemaphoreType.DMA,
        ],
    )
    def gather_k(table_hbm, idx_hbm, out_hbm, idx_v, rows_v, sem):
        wid = lax.axis_index("s") * nc + lax.axis_index("c")
        base = wid * b_per_w
        pltpu.sync_copy(idx_hbm.at[pl.ds(base, b_per_w)], idx_v)
        pltpu.async_copy(table_hbm.at[idx_v], rows_v, sem).wait()
        pltpu.sync_copy(rows_v, out_hbm.at[pl.ds(base, b_per_w)])

    return gather_k


def kernel(x, labels, centers):
    num_classes, cpc, d = centers.shape
    b = x.shape[0]
    k = num_classes * cpc
    allc = centers.reshape(k, d).astype(x.dtype)
    x2 = jnp.sum(x * x, axis=1, keepdims=True)
    c2 = jnp.sum(allc * allc, axis=1)[None, :]
    iotaf = jnp.arange(k, dtype=jnp.float32)[None, :]
    nb = b // BM
    nk = k // KC

    out_shapes = [
        jax.ShapeDtypeStruct((b, 1), jnp.int32),
        jax.ShapeDtypeStruct((b, 1), jnp.float32),
        jax.ShapeDtypeStruct((b, 1), jnp.int32),
        jax.ShapeDtypeStruct((b, 1), jnp.int32),
    ]
    grid_spec = pl.GridSpec(
        grid=(nb,),
        in_specs=[
            pl.BlockSpec((BM, d), lambda i: (i, 0)),
            pl.BlockSpec((k, d), lambda i: (0, 0)),
            pl.BlockSpec((BM, 1), lambda i: (i, 0)),
            pl.BlockSpec((1, k), lambda i: (0, 0)),
            pl.BlockSpec((1, k), lambda i: (0, 0)),
        ],
        out_specs=[
            pl.BlockSpec((BM, 1), lambda i: (i, 0)),
            pl.BlockSpec((BM, 1), lambda i: (i, 0)),
            pl.BlockSpec((BM, 1), lambda i: (i, 0)),
            pl.BlockSpec((BM, 1), lambda i: (i, 0)),
        ],
    )
    idx, mind, cls, clu = pl.pallas_call(
        functools.partial(_vq_body, nk, cpc=cpc),
        grid_spec=grid_spec,
        out_shape=out_shapes,
    )(x, allc, x2, c2, iotaf)

    quant = _make_sc_gather(k, d, b)(allc, idx.reshape(b))

    return (quant, clu.reshape(b), mind.reshape(b), cls.reshape(b),
            centers, labels)


# speedup vs baseline: 1.0928x; 1.0350x over previous
import functools

import jax
import jax.numpy as jnp
from jax import lax
from jax.experimental import pallas as pl
from jax.experimental.pallas import tpu as pltpu
from jax.experimental.pallas import tpu_sc as plsc


BM = 256
KC = 1024
BLK = 128


def _vq_body(nk, x_ref, ct_ref, x2_ref, c2_ref, iotaf_ref, out_ref):
    x = -2.0 * x_ref[...]
    x2 = x2_ref[...]

    m = jnp.full((BM, BLK), jnp.inf, dtype=jnp.float32)
    f = jnp.zeros((BM, BLK), dtype=jnp.float32)

    for j in range(nk):
        ctc = ct_ref[j * KC:(j + 1) * KC, :]
        g = lax.dot_general(x, ctc, (((1,), (1,)), ((), ())),
                            preferred_element_type=jnp.float32)
        for b in range(KC // BLK):
            lo = j * KC + b * BLK
            c2 = c2_ref[:, lo:lo + BLK]
            iota_f = iotaf_ref[:, lo:lo + BLK]
            d2 = (x2 + c2) + g[:, b * BLK:(b + 1) * BLK]
            lt = d2 < m
            m = jnp.where(lt, d2, m)
            f = jnp.where(lt, iota_f, f)

    mv = jnp.min(m, axis=1, keepdims=True)
    big = float(nk * KC)
    li = jnp.min(jnp.where(m == mv, f, big), axis=1, keepdims=True)
    out_ref[...] = jnp.concatenate([mv, li], axis=1)


def _make_sc_gather(k, d, b):
    info = plsc.get_sparse_core_info()
    nc, ns = info.num_cores, info.num_subcores
    nw = nc * ns
    b_per_w = b // nw
    mesh = plsc.VectorSubcoreMesh(core_axis_name="c", subcore_axis_name="s")

    @functools.partial(
        pl.kernel, mesh=mesh,
        compiler_params=pltpu.CompilerParams(use_tc_tiling_on_sc=False),
        out_type=jax.ShapeDtypeStruct((b, d), jnp.float32),
        scratch_types=[
            pltpu.VMEM((b_per_w,), jnp.int32),
            pltpu.VMEM((b_per_w, d), jnp.float32),
            pltpu.SemaphoreType.DMA,
        ],
    )
    def gather_k(table_hbm, idx_hbm, out_hbm, idx_v, rows_v, sem):
        wid = lax.axis_index("s") * nc + lax.axis_index("c")
        base = wid * b_per_w
        pltpu.sync_copy(idx_hbm.at[pl.ds(base, b_per_w)], idx_v)
        pltpu.async_copy(table_hbm.at[idx_v], rows_v, sem).wait()
        pltpu.sync_copy(rows_v, out_hbm.at[pl.ds(base, b_per_w)])

    return gather_k


def kernel(x, labels, centers):
    num_classes, cpc, d = centers.shape
    b = x.shape[0]
    k = num_classes * cpc
    allc = centers.reshape(k, d).astype(x.dtype)
    x2 = jnp.sum(x * x, axis=1, keepdims=True)
    c2 = jnp.sum(allc * allc, axis=1)[None, :]
    iotaf = jnp.arange(k, dtype=jnp.float32)[None, :]
    nb = b // BM
    nk = k // KC

    out_shapes = jax.ShapeDtypeStruct((b, 2), jnp.float32)
    grid_spec = pl.GridSpec(
        grid=(nb,),
        in_specs=[
            pl.BlockSpec((BM, d), lambda i: (i, 0)),
            pl.BlockSpec((k, d), lambda i: (0, 0)),
            pl.BlockSpec((BM, 1), lambda i: (i, 0)),
            pl.BlockSpec((1, k), lambda i: (0, 0)),
            pl.BlockSpec((1, k), lambda i: (0, 0)),
        ],
        out_specs=pl.BlockSpec((BM, 2), lambda i: (i, 0)),
    )
    packed = pl.pallas_call(
        functools.partial(_vq_body, nk),
        grid_spec=grid_spec,
        out_shape=out_shapes,
    )(x, allc, x2, c2, iotaf)

    mind = jnp.sqrt(jnp.maximum(packed[:, 0], 0.0))
    idx = packed[:, 1].astype(jnp.int32)
    cls = idx // cpc
    clu = idx % cpc
    quant = _make_sc_gather(k, d, b)(allc, idx)

    return (quant, clu, mind, cls, centers, labels)


# speedup vs baseline: 1.0944x; 1.0015x over previous
import functools

import jax
import jax.numpy as jnp
from jax import lax
from jax.experimental import pallas as pl
from jax.experimental.pallas import tpu as pltpu
from jax.experimental.pallas import tpu_sc as plsc


BM = 256
KC = 1024
BLK = 128


def _vq_body(nk, xt_ref, ct_ref, x2_ref, c2_ref, iotaf_ref, out_ref):
    xt = -2.0 * xt_ref[...]

    m = jnp.full((BM, BLK), jnp.inf, dtype=jnp.float32)
    f = jnp.zeros((BM, BLK), dtype=jnp.float32)
    x2 = x2_ref[...]

    for j in range(nk):
        ctc = ct_ref[j * KC:(j + 1) * KC, :]
        g = lax.dot_general(xt, ctc, (((0,), (1,)), ((), ())),
                            preferred_element_type=jnp.float32)
        for b in range(KC // BLK):
            lo = j * KC + b * BLK
            c2 = c2_ref[:, lo:lo + BLK]
            iota_f = iotaf_ref[:, lo:lo + BLK]
            d2 = (x2 + c2) + g[:, b * BLK:(b + 1) * BLK]
            lt = d2 < m
            m = jnp.where(lt, d2, m)
            f = jnp.where(lt, iota_f, f)

    mv = jnp.min(m, axis=1, keepdims=True)
    big = float(nk * KC)
    li = jnp.min(jnp.where(m == mv, f, big), axis=1, keepdims=True)
    out_ref[...] = jnp.concatenate([mv, li], axis=1)


def _make_sc_gather(k, d, b):
    info = plsc.get_sparse_core_info()
    nc, ns = info.num_cores, info.num_subcores
    nw = nc * ns
    b_per_w = b // nw
    mesh = plsc.VectorSubcoreMesh(core_axis_name="c", subcore_axis_name="s")

    @functools.partial(
        pl.kernel, mesh=mesh,
        compiler_params=pltpu.CompilerParams(use_tc_tiling_on_sc=False),
        out_type=jax.ShapeDtypeStruct((b, d), jnp.float32),
        scratch_types=[
            pltpu.VMEM((b_per_w,), jnp.int32),
            pltpu.VMEM((b_per_w, d), jnp.float32),
            pltpu.SemaphoreType.DMA,
        ],
    )
    def gather_k(table_hbm, idx_hbm, out_hbm, idx_v, rows_v, sem):
        wid = lax.axis_index("s") * nc + lax.axis_index("c")
        base = wid * b_per_w
        pltpu.sync_copy(idx_hbm.at[pl.ds(base, b_per_w)], idx_v)
        pltpu.async_copy(table_hbm.at[idx_v], rows_v, sem).wait()
        pltpu.sync_copy(rows_v, out_hbm.at[pl.ds(base, b_per_w)])

    return gather_k


def kernel(x, labels, centers):
    num_classes, cpc, d = centers.shape
    b = x.shape[0]
    k = num_classes * cpc
    allc = centers.reshape(k, d).astype(x.dtype)
    x2 = jnp.sum(x * x, axis=1, keepdims=True)
    c2 = jnp.sum(allc * allc, axis=1)[None, :]
    iotaf = jnp.arange(k, dtype=jnp.float32)[None, :]
    nb = b // BM
    nk = k // KC

    out_shapes = jax.ShapeDtypeStruct((b, 2), jnp.float32)
    grid_spec = pl.GridSpec(
        grid=(nb,),
        in_specs=[
            pl.BlockSpec((d, BM), lambda i: (0, i)),
            pl.BlockSpec((k, d), lambda i: (0, 0)),
            pl.BlockSpec((BM, 1), lambda i: (i, 0)),
            pl.BlockSpec((1, k), lambda i: (0, 0)),
            pl.BlockSpec((1, k), lambda i: (0, 0)),
        ],
        out_specs=pl.BlockSpec((BM, 2), lambda i: (i, 0)),
    )
    packed = pl.pallas_call(
        functools.partial(_vq_body, nk),
        grid_spec=grid_spec,
        out_shape=out_shapes,
    )(x.T, allc, x2, c2, iotaf)

    mind = jnp.sqrt(jnp.maximum(packed[:, 0], 0.0))
    idx = packed[:, 1].astype(jnp.int32)
    cls = idx // cpc
    clu = idx % cpc
    quant = _make_sc_gather(k, d, b)(allc, idx)

    return (quant, clu, mind, cls, centers, labels)


# speedup vs baseline: 1.1751x; 1.0737x over previous
import functools

import jax
import jax.numpy as jnp
from jax import lax
from jax.experimental import pallas as pl
from jax.experimental.pallas import tpu as pltpu
from jax.experimental.pallas import tpu_sc as plsc


BM = 256
KC = 1024
BLK = 128


def _vq_body(nk, xt_ref, ct_ref, x2_ref, c2_ref, iotaf_ref, out_ref):
    xt = -2.0 * xt_ref[...]

    m = jnp.full((BM, BLK), jnp.inf, dtype=jnp.float32)
    f = jnp.zeros((BM, BLK), dtype=jnp.float32)
    x2 = x2_ref[...].T

    for j in range(nk):
        ctc = ct_ref[j * KC:(j + 1) * KC, :]
        g = lax.dot_general(xt, ctc, (((0,), (1,)), ((), ())),
                            preferred_element_type=jnp.float32)
        for b in range(KC // BLK):
            lo = j * KC + b * BLK
            c2 = c2_ref[:, lo:lo + BLK]
            iota_f = iotaf_ref[:, lo:lo + BLK]
            d2 = (x2 + c2) + g[:, b * BLK:(b + 1) * BLK]
            lt = d2 < m
            m = jnp.where(lt, d2, m)
            f = jnp.where(lt, iota_f, f)

    mv = jnp.min(m, axis=1, keepdims=True)
    big = float(nk * KC)
    li = jnp.min(jnp.where(m == mv, f, big), axis=1, keepdims=True)
    packed = jnp.concatenate([mv, li], axis=1)
    out_ref[...] = packed.T


def _make_sc_gather(k, d, b):
    info = plsc.get_sparse_core_info()
    nc, ns = info.num_cores, info.num_subcores
    nw = nc * ns
    b_per_w = b // nw
    mesh = plsc.VectorSubcoreMesh(core_axis_name="c", subcore_axis_name="s")

    @functools.partial(
        pl.kernel, mesh=mesh,
        compiler_params=pltpu.CompilerParams(use_tc_tiling_on_sc=False),
        out_type=jax.ShapeDtypeStruct((b, d), jnp.float32),
        scratch_types=[
            pltpu.VMEM((b_per_w,), jnp.int32),
            pltpu.VMEM((b_per_w, d), jnp.float32),
            pltpu.SemaphoreType.DMA,
        ],
    )
    def gather_k(table_hbm, idx_hbm, out_hbm, idx_v, rows_v, sem):
        wid = lax.axis_index("s") * nc + lax.axis_index("c")
        base = wid * b_per_w
        pltpu.sync_copy(idx_hbm.at[pl.ds(base, b_per_w)], idx_v)
        pltpu.async_copy(table_hbm.at[idx_v], rows_v, sem).wait()
        pltpu.sync_copy(rows_v, out_hbm.at[pl.ds(base, b_per_w)])

    return gather_k


def kernel(x, labels, centers):
    num_classes, cpc, d = centers.shape
    b = x.shape[0]
    k = num_classes * cpc
    allc = centers.reshape(k, d).astype(x.dtype)
    x2 = jnp.sum(x * x, axis=1, keepdims=True).T
    c2 = jnp.sum(allc * allc, axis=1)[None, :]
    iotaf = jnp.arange(k, dtype=jnp.float32)[None, :]
    nb = b // BM
    nk = k // KC

    out_shapes = jax.ShapeDtypeStruct((2, b), jnp.float32)
    grid_spec = pl.GridSpec(
        grid=(nb,),
        in_specs=[
            pl.BlockSpec((d, BM), lambda i: (0, i)),
            pl.BlockSpec((k, d), lambda i: (0, 0)),
            pl.BlockSpec((1, BM), lambda i: (0, i)),
            pl.BlockSpec((1, k), lambda i: (0, 0)),
            pl.BlockSpec((1, k), lambda i: (0, 0)),
        ],
        out_specs=pl.BlockSpec((2, BM), lambda i: (0, i)),
    )
    packed = pl.pallas_call(
        functools.partial(_vq_body, nk),
        grid_spec=grid_spec,
        out_shape=out_shapes,
    )(x.T, allc, x2, c2, iotaf)

    mind = jnp.sqrt(jnp.maximum(packed[0, :], 0.0))
    idx = packed[1, :].astype(jnp.int32)
    cls = idx // cpc
    clu = idx % cpc
    quant = _make_sc_gather(k, d, b)(allc, idx)

    return (quant, clu, mind, cls, centers, labels)


# speedup vs baseline: 1.1755x; 1.0004x over previous
import functools

import jax
import jax.numpy as jnp
from jax import lax
from jax.experimental import pallas as pl
from jax.experimental.pallas import tpu as pltpu
from jax.experimental.pallas import tpu_sc as plsc


BM = 256
KC = 1024
BLK = 128


def _vq_body(nk, xt_ref, ct_ref, x2_ref, c2_ref, iotaf_ref, out_ref):
    xt = -2.0 * xt_ref[...]

    m = jnp.full((BM, BLK), jnp.inf, dtype=jnp.float32)
    f = jnp.zeros((BM, BLK), dtype=jnp.float32)
    x2 = x2_ref[...].T

    for j in range(nk):
        ctc = ct_ref[j * KC:(j + 1) * KC, :]
        g = lax.dot_general(xt, ctc, (((0,), (1,)), ((), ())),
                            preferred_element_type=jnp.float32)
        for b in range(KC // BLK):
            lo = j * KC + b * BLK
            c2 = c2_ref[:, lo:lo + BLK]
            iota_f = iotaf_ref[:, lo:lo + BLK]
            d2 = (x2 + c2) + g[:, b * BLK:(b + 1) * BLK]
            lt = d2 < m
            m = jnp.where(lt, d2, m)
            f = jnp.where(lt, iota_f, f)

    mv = jnp.min(m, axis=1, keepdims=True)
    big = float(nk * KC)
    li = jnp.min(jnp.where(m == mv, f, big), axis=1, keepdims=True)
    packed = jnp.concatenate([mv, li], axis=1)
    out_ref[...] = packed.T


def _make_sc_gather(k, d, b):
    info = plsc.get_sparse_core_info()
    nc, ns = info.num_cores, info.num_subcores
    nw = nc * ns
    b_per_w = b // nw
    mesh = plsc.VectorSubcoreMesh(core_axis_name="c", subcore_axis_name="s")

    half = b_per_w // 2

    @functools.partial(
        pl.kernel, mesh=mesh,
        compiler_params=pltpu.CompilerParams(use_tc_tiling_on_sc=False),
        out_type=jax.ShapeDtypeStruct((b, d), jnp.float32),
        scratch_types=[
            pltpu.VMEM((2, half), jnp.int32),
            pltpu.VMEM((2, half, d), jnp.float32),
            pltpu.SemaphoreType.DMA,
            pltpu.SemaphoreType.DMA,
            pltpu.SemaphoreType.DMA,
        ],
    )
    def gather_k(table_hbm, idx_hbm, out_hbm, idx_v, rows_v, g0s, g1s, w0s):
        wid = lax.axis_index("s") * nc + lax.axis_index("c")
        base = wid * b_per_w
        pltpu.sync_copy(idx_hbm.at[pl.ds(base, half)], idx_v.at[0])
        g0 = pltpu.async_copy(table_hbm.at[idx_v.at[0]], rows_v.at[0], g0s)
        pltpu.sync_copy(idx_hbm.at[pl.ds(base + half, half)], idx_v.at[1])
        g1 = pltpu.async_copy(table_hbm.at[idx_v.at[1]], rows_v.at[1], g1s)
        g0.wait()
        w0 = pltpu.async_copy(rows_v.at[0], out_hbm.at[pl.ds(base, half)], w0s)
        g1.wait()
        pltpu.sync_copy(rows_v.at[1], out_hbm.at[pl.ds(base + half, half)])
        w0.wait()

    return gather_k


def kernel(x, labels, centers):
    num_classes, cpc, d = centers.shape
    b = x.shape[0]
    k = num_classes * cpc
    allc = centers.reshape(k, d).astype(x.dtype)
    x2 = jnp.sum(x * x, axis=1, keepdims=True).T
    c2 = jnp.sum(allc * allc, axis=1)[None, :]
    iotaf = jnp.arange(k, dtype=jnp.float32)[None, :]
    nb = b // BM
    nk = k // KC

    out_shapes = jax.ShapeDtypeStruct((2, b), jnp.float32)
    grid_spec = pl.GridSpec(
        grid=(nb,),
        in_specs=[
            pl.BlockSpec((d, BM), lambda i: (0, i)),
            pl.BlockSpec((k, d), lambda i: (0, 0)),
            pl.BlockSpec((1, BM), lambda i: (0, i)),
            pl.BlockSpec((1, k), lambda i: (0, 0)),
            pl.BlockSpec((1, k), lambda i: (0, 0)),
        ],
        out_specs=pl.BlockSpec((2, BM), lambda i: (0, i)),
    )
    packed = pl.pallas_call(
        functools.partial(_vq_body, nk),
        grid_spec=grid_spec,
        out_shape=out_shapes,
    )(x.T, allc, x2, c2, iotaf)

    mind = jnp.sqrt(jnp.maximum(packed[0, :], 0.0))
    idx = packed[1, :].astype(jnp.int32)
    cls = idx // cpc
    clu = idx % cpc
    quant = _make_sc_gather(k, d, b)(allc, idx)

    return (quant, clu, mind, cls, centers, labels)


# speedup vs baseline: 1.2016x; 1.0222x over previous
import functools

import jax
import jax.numpy as jnp
from jax import lax
from jax.experimental import pallas as pl
from jax.experimental.pallas import tpu as pltpu
from jax.experimental.pallas import tpu_sc as plsc


BM = 512
KC = 1024
BLK = 128


def _vq_body(nk, xt_ref, ct_ref, x2_ref, c2_ref, iotaf_ref, out_ref):
    xt = -2.0 * xt_ref[...]

    m = jnp.full((BM, BLK), jnp.inf, dtype=jnp.float32)
    f = jnp.zeros((BM, BLK), dtype=jnp.float32)
    x2 = x2_ref[...].T

    for j in range(nk):
        ctc = ct_ref[j * KC:(j + 1) * KC, :]
        g = lax.dot_general(xt, ctc, (((0,), (1,)), ((), ())),
                            preferred_element_type=jnp.float32)
        for b in range(KC // BLK):
            lo = j * KC + b * BLK
            c2 = c2_ref[:, lo:lo + BLK]
            iota_f = iotaf_ref[:, lo:lo + BLK]
            d2 = (x2 + c2) + g[:, b * BLK:(b + 1) * BLK]
            lt = d2 < m
            m = jnp.where(lt, d2, m)
            f = jnp.where(lt, iota_f, f)

    mv = jnp.min(m, axis=1, keepdims=True)
    big = float(nk * KC)
    li = jnp.min(jnp.where(m == mv, f, big), axis=1, keepdims=True)
    packed = jnp.concatenate([mv, li], axis=1)
    out_ref[...] = packed.T


def _make_sc_gather(k, d, b):
    info = plsc.get_sparse_core_info()
    nc, ns = info.num_cores, info.num_subcores
    nw = nc * ns
    b_per_w = b // nw
    mesh = plsc.VectorSubcoreMesh(core_axis_name="c", subcore_axis_name="s")

    half = b_per_w // 2

    @functools.partial(
        pl.kernel, mesh=mesh,
        compiler_params=pltpu.CompilerParams(use_tc_tiling_on_sc=False),
        out_type=jax.ShapeDtypeStruct((b, d), jnp.float32),
        scratch_types=[
            pltpu.VMEM((2, half), jnp.int32),
            pltpu.VMEM((2, half, d), jnp.float32),
            pltpu.SemaphoreType.DMA,
            pltpu.SemaphoreType.DMA,
            pltpu.SemaphoreType.DMA,
        ],
    )
    def gather_k(table_hbm, idx_hbm, out_hbm, idx_v, rows_v, g0s, g1s, w0s):
        wid = lax.axis_index("s") * nc + lax.axis_index("c")
        base = wid * b_per_w
        pltpu.sync_copy(idx_hbm.at[pl.ds(base, half)], idx_v.at[0])
        g0 = pltpu.async_copy(table_hbm.at[idx_v.at[0]], rows_v.at[0], g0s)
        pltpu.sync_copy(idx_hbm.at[pl.ds(base + half, half)], idx_v.at[1])
        g1 = pltpu.async_copy(table_hbm.at[idx_v.at[1]], rows_v.at[1], g1s)
        g0.wait()
        w0 = pltpu.async_copy(rows_v.at[0], out_hbm.at[pl.ds(base, half)], w0s)
        g1.wait()
        pltpu.sync_copy(rows_v.at[1], out_hbm.at[pl.ds(base + half, half)])
        w0.wait()

    return gather_k


def kernel(x, labels, centers):
    num_classes, cpc, d = centers.shape
    b = x.shape[0]
    k = num_classes * cpc
    allc = centers.reshape(k, d).astype(x.dtype)
    x2 = jnp.sum(x * x, axis=1, keepdims=True).T
    c2 = jnp.sum(allc * allc, axis=1)[None, :]
    iotaf = jnp.arange(k, dtype=jnp.float32)[None, :]
    nb = b // BM
    nk = k // KC

    out_shapes = jax.ShapeDtypeStruct((2, b), jnp.float32)
    grid_spec = pl.GridSpec(
        grid=(nb,),
        in_specs=[
            pl.BlockSpec((d, BM), lambda i: (0, i)),
            pl.BlockSpec((k, d), lambda i: (0, 0)),
            pl.BlockSpec((1, BM), lambda i: (0, i)),
            pl.BlockSpec((1, k), lambda i: (0, 0)),
            pl.BlockSpec((1, k), lambda i: (0, 0)),
        ],
        out_specs=pl.BlockSpec((2, BM), lambda i: (0, i)),
    )
    packed = pl.pallas_call(
        functools.partial(_vq_body, nk),
        grid_spec=grid_spec,
        out_shape=out_shapes,
    )(x.T, allc, x2, c2, iotaf)

    mind = jnp.sqrt(jnp.maximum(packed[0, :], 0.0))
    idx = packed[1, :].astype(jnp.int32)
    cls = idx // cpc
    clu = idx % cpc
    quant = _make_sc_gather(k, d, b)(allc, idx)

    return (quant, clu, mind, cls, centers, labels)


# speedup vs baseline: 1.2398x; 1.0318x over previous
import functools

import jax
import jax.numpy as jnp
from jax import lax
from jax.experimental import pallas as pl
from jax.experimental.pallas import tpu as pltpu
from jax.experimental.pallas import tpu_sc as plsc


BM = 1024
KC = 1024
BLK = 128


def _vq_body(nk, xt_ref, ct_ref, x2_ref, c2_ref, iotaf_ref, out_ref):
    xt = -2.0 * xt_ref[...]

    m = jnp.full((BM, BLK), jnp.inf, dtype=jnp.float32)
    f = jnp.zeros((BM, BLK), dtype=jnp.float32)
    x2 = x2_ref[...].T

    for j in range(nk):
        ctc = ct_ref[j * KC:(j + 1) * KC, :]
        g = lax.dot_general(xt, ctc, (((0,), (1,)), ((), ())),
                            preferred_element_type=jnp.float32)
        for b in range(KC // BLK):
            lo = j * KC + b * BLK
            c2 = c2_ref[:, lo:lo + BLK]
            iota_f = iotaf_ref[:, lo:lo + BLK]
            d2 = (x2 + c2) + g[:, b * BLK:(b + 1) * BLK]
            lt = d2 < m
            m = jnp.where(lt, d2, m)
            f = jnp.where(lt, iota_f, f)

    mv = jnp.min(m, axis=1, keepdims=True)
    big = float(nk * KC)
    li = jnp.min(jnp.where(m == mv, f, big), axis=1, keepdims=True)
    packed = jnp.concatenate([mv, li], axis=1)
    out_ref[...] = packed.T


def _make_sc_gather(k, d, b):
    info = plsc.get_sparse_core_info()
    nc, ns = info.num_cores, info.num_subcores
    nw = nc * ns
    b_per_w = b // nw
    mesh = plsc.VectorSubcoreMesh(core_axis_name="c", subcore_axis_name="s")

    half = b_per_w // 2

    @functools.partial(
        pl.kernel, mesh=mesh,
        compiler_params=pltpu.CompilerParams(use_tc_tiling_on_sc=False),
        out_type=jax.ShapeDtypeStruct((b, d), jnp.float32),
        scratch_types=[
            pltpu.VMEM((2, half), jnp.int32),
            pltpu.VMEM((2, half, d), jnp.float32),
            pltpu.SemaphoreType.DMA,
            pltpu.SemaphoreType.DMA,
            pltpu.SemaphoreType.DMA,
        ],
    )
    def gather_k(table_hbm, idx_hbm, out_hbm, idx_v, rows_v, g0s, g1s, w0s):
        wid = lax.axis_index("s") * nc + lax.axis_index("c")
        base = wid * b_per_w
        pltpu.sync_copy(idx_hbm.at[pl.ds(base, half)], idx_v.at[0])
        g0 = pltpu.async_copy(table_hbm.at[idx_v.at[0]], rows_v.at[0], g0s)
        pltpu.sync_copy(idx_hbm.at[pl.ds(base + half, half)], idx_v.at[1])
        g1 = pltpu.async_copy(table_hbm.at[idx_v.at[1]], rows_v.at[1], g1s)
        g0.wait()
        w0 = pltpu.async_copy(rows_v.at[0], out_hbm.at[pl.ds(base, half)], w0s)
        g1.wait()
        pltpu.sync_copy(rows_v.at[1], out_hbm.at[pl.ds(base + half, half)])
        w0.wait()

    return gather_k


def kernel(x, labels, centers):
    num_classes, cpc, d = centers.shape
    b = x.shape[0]
    k = num_classes * cpc
    allc = centers.reshape(k, d).astype(x.dtype)
    x2 = jnp.sum(x * x, axis=1, keepdims=True).T
    c2 = jnp.sum(allc * allc, axis=1)[None, :]
    iotaf = jnp.arange(k, dtype=jnp.float32)[None, :]
    nb = b // BM
    nk = k // KC

    out_shapes = jax.ShapeDtypeStruct((2, b), jnp.float32)
    grid_spec = pl.GridSpec(
        grid=(nb,),
        in_specs=[
            pl.BlockSpec((d, BM), lambda i: (0, i)),
            pl.BlockSpec((k, d), lambda i: (0, 0)),
            pl.BlockSpec((1, BM), lambda i: (0, i)),
            pl.BlockSpec((1, k), lambda i: (0, 0)),
            pl.BlockSpec((1, k), lambda i: (0, 0)),
        ],
        out_specs=pl.BlockSpec((2, BM), lambda i: (0, i)),
    )
    packed = pl.pallas_call(
        functools.partial(_vq_body, nk),
        grid_spec=grid_spec,
        out_shape=out_shapes,
    )(x.T, allc, x2, c2, iotaf)

    mind = jnp.sqrt(jnp.maximum(packed[0, :], 0.0))
    idx = packed[1, :].astype(jnp.int32)
    cls = idx // cpc
    clu = idx % cpc
    quant = _make_sc_gather(k, d, b)(allc, idx)

    return (quant, clu, mind, cls, centers, labels)
